# baseline (speedup 1.0000x reference)
import functools

import jax
import jax.numpy as jnp
from jax import lax
from jax.experimental import pallas as pl
from jax.experimental.pallas import tpu as pltpu
from jax.experimental.pallas import tpu_sc as plsc

B = 16384
D = 64
NC, NS = 2, 16
NW = NC * NS
BPW = B // NW


@functools.lru_cache(maxsize=None)
def _gather_kernel():
    mesh = plsc.VectorSubcoreMesh(core_axis_name="c", subcore_axis_name="s")

    @functools.partial(
        pl.kernel,
        out_type=(
            jax.ShapeDtypeStruct((B, D), jnp.bfloat16),
            jax.ShapeDtypeStruct((B, D), jnp.bfloat16),
        ),
        mesh=mesh,
        scratch_types=[
            pltpu.VMEM((BPW,), jnp.int32),
            pltpu.VMEM((BPW,), jnp.int32),
            pltpu.VMEM((BPW, D), jnp.bfloat16),
            pltpu.VMEM((BPW, D), jnp.bfloat16),
            pltpu.SemaphoreType.DMA,
            pltpu.SemaphoreType.DMA,
        ],
        compiler_params=pltpu.CompilerParams(use_tc_tiling_on_sc=False),
    )
    def gather(user_hbm, item_hbm, ut_hbm, it_hbm, uout_hbm, iout_hbm,
               uidx_v, iidx_v, urows_v, irows_v, usem, isem):
        wid = lax.axis_index("s") * NC + lax.axis_index("c")
        base = wid * BPW
        pltpu.sync_copy(user_hbm.at[pl.ds(base, BPW)], uidx_v)
        pltpu.sync_copy(item_hbm.at[pl.ds(base, BPW)], iidx_v)
        cu = pltpu.async_copy(ut_hbm.at[uidx_v], urows_v, usem)
        ci = pltpu.async_copy(it_hbm.at[iidx_v], irows_v, isem)
        cu.wait()
        ci.wait()
        pltpu.sync_copy(urows_v, uout_hbm.at[pl.ds(base, BPW)])
        pltpu.sync_copy(irows_v, iout_hbm.at[pl.ds(base, BPW)])

    return gather


def _mlp_body(u_ref, i_ref, w1u_ref, w1i_ref, b1_ref, w2_ref, b2_ref,
              w3_ref, b3_ref, o_ref):
    h = (jnp.dot(u_ref[...], w1u_ref[...], preferred_element_type=jnp.float32)
         + jnp.dot(i_ref[...], w1i_ref[...], preferred_element_type=jnp.float32)
         + b1_ref[...])
    h = jnp.maximum(h, 0.0).astype(jnp.bfloat16)
    h = jnp.dot(h, w2_ref[...], preferred_element_type=jnp.float32) + b2_ref[...]
    h = jnp.maximum(h, 0.0)
    y = jnp.sum(h * w3_ref[...], axis=1, keepdims=True) + b3_ref[...]
    o_ref[...] = jax.nn.sigmoid(y)


def _mlp(u, i, W1, b1, W2, b2, W3, b3, block_b=2048):
    w1t = W1.T.astype(jnp.bfloat16)
    w1u, w1i = w1t[:D], w1t[D:]
    w2t = W2.T.astype(jnp.bfloat16)
    w3 = W3.reshape(1, 32)
    full = lambda shape: pl.BlockSpec(shape, lambda b: (0, 0))
    return pl.pallas_call(
        _mlp_body,
        grid=(B // block_b,),
        in_specs=[
            pl.BlockSpec((block_b, D), lambda b: (b, 0)),
            pl.BlockSpec((block_b, D), lambda b: (b, 0)),
            full((D, 64)),
            full((D, 64)),
            full((1, 64)),
            full((64, 32)),
            full((1, 32)),
            full((1, 32)),
            full((1, 1)),
        ],
        out_specs=pl.BlockSpec((block_b, 1), lambda b: (b, 0)),
        out_shape=jax.ShapeDtypeStruct((B, 1), jnp.float32),
    )(u, i, w1u, w1i, b1.reshape(1, 64), w2t, b2.reshape(1, 32), w3,
      b3.reshape(1, 1))


def kernel(user, item, user_table, item_table, W1, b1, W2, b2, W3, b3):
    u_emb, i_emb = _gather_kernel()(user.astype(jnp.int32),
                                    item.astype(jnp.int32),
                                    user_table.astype(jnp.bfloat16),
                                    item_table.astype(jnp.bfloat16))
    return _mlp(u_emb, i_emb, W1, b1, W2, b2, W3, b3)


# speedup vs baseline: 1.9122x; 1.9122x over previous
import functools

import jax
import jax.numpy as jnp
from jax import lax
from jax.experimental import pallas as pl
from jax.experimental.pallas import tpu as pltpu
from jax.experimental.pallas import tpu_sc as plsc

B = 16384
D = 64
NV = 100000
NC, NS = 2, 16
NW = NC * NS
BPW = B // NW
RBLK = 2048


def _repack_body(ut_ref, it_ref, o_ref):
    o_ref[:, 0:D] = jnp.transpose(ut_ref[...], (1, 0))
    o_ref[:, D:2 * D] = jnp.transpose(it_ref[...], (1, 0))


def _repack(ut_t, it_t):
    return pl.pallas_call(
        _repack_body,
        grid=(pl.cdiv(NV, RBLK),),
        in_specs=[
            pl.BlockSpec((D, RBLK), lambda j: (0, j)),
            pl.BlockSpec((D, RBLK), lambda j: (0, j)),
        ],
        out_specs=pl.BlockSpec((RBLK, 2 * D), lambda j: (j, 0)),
        out_shape=jax.ShapeDtypeStruct((NV, 2 * D), jnp.float32),
    )(ut_t, it_t)


@functools.lru_cache(maxsize=None)
def _gather_kernel():
    mesh = plsc.VectorSubcoreMesh(core_axis_name="c", subcore_axis_name="s")

    @functools.partial(
        pl.kernel,
        out_type=(
            jax.ShapeDtypeStruct((B, 2 * D), jnp.float32),
            jax.ShapeDtypeStruct((B, 2 * D), jnp.float32),
        ),
        mesh=mesh,
        scratch_types=[
            pltpu.VMEM((BPW,), jnp.int32),
            pltpu.VMEM((BPW,), jnp.int32),
            pltpu.VMEM((BPW, 2 * D), jnp.float32),
            pltpu.SemaphoreType.DMA,
        ],
        compiler_params=pltpu.CompilerParams(use_tc_tiling_on_sc=False),
    )
    def gather(user_hbm, item_hbm, tab_hbm, uout_hbm, iout_hbm,
               uidx_v, iidx_v, rows_v, sem):
        wid = lax.axis_index("s") * NC + lax.axis_index("c")
        base = wid * BPW
        pltpu.sync_copy(user_hbm.at[pl.ds(base, BPW)], uidx_v)
        pltpu.sync_copy(item_hbm.at[pl.ds(base, BPW)], iidx_v)
        pltpu.async_copy(tab_hbm.at[uidx_v], rows_v, sem).wait()
        pltpu.sync_copy(rows_v, uout_hbm.at[pl.ds(base, BPW)])
        pltpu.async_copy(tab_hbm.at[iidx_v], rows_v, sem).wait()
        pltpu.sync_copy(rows_v, iout_hbm.at[pl.ds(base, BPW)])

    return gather


def _mlp_body(u_ref, i_ref, w1u_ref, w1i_ref, b1_ref, w2_ref, b2_ref,
              w3_ref, b3_ref, o_ref):
    u = u_ref[:, 0:D].astype(jnp.bfloat16)
    i = i_ref[:, D:2 * D].astype(jnp.bfloat16)
    h = (jnp.dot(u, w1u_ref[...], preferred_element_type=jnp.float32)
         + jnp.dot(i, w1i_ref[...], preferred_element_type=jnp.float32)
         + b1_ref[...])
    h = jnp.maximum(h, 0.0).astype(jnp.bfloat16)
    h = jnp.dot(h, w2_ref[...], preferred_element_type=jnp.float32) + b2_ref[...]
    h = jnp.maximum(h, 0.0)
    y = jnp.sum(h * w3_ref[...], axis=1, keepdims=True) + b3_ref[...]
    o_ref[...] = jax.nn.sigmoid(y)


def _mlp(ug, ig, W1, b1, W2, b2, W3, b3, block_b=2048):
    w1t = W1.T.astype(jnp.bfloat16)
    w1u, w1i = w1t[:D], w1t[D:]
    w2t = W2.T.astype(jnp.bfloat16)
    w3 = W3.reshape(1, 32)
    full = lambda shape: pl.BlockSpec(shape, lambda b: (0, 0))
    return pl.pallas_call(
        _mlp_body,
        grid=(B // block_b,),
        in_specs=[
            pl.BlockSpec((block_b, 2 * D), lambda b: (b, 0)),
            pl.BlockSpec((block_b, 2 * D), lambda b: (b, 0)),
            full((D, 64)),
            full((D, 64)),
            full((1, 64)),
            full((64, 32)),
            full((1, 32)),
            full((1, 32)),
            full((1, 1)),
        ],
        out_specs=pl.BlockSpec((block_b, 1), lambda b: (b, 0)),
        out_shape=jax.ShapeDtypeStruct((B, 1), jnp.float32),
    )(ug, ig, w1u, w1i, b1.reshape(1, 64), w2t, b2.reshape(1, 32), w3,
      b3.reshape(1, 1))


def kernel(user, item, user_table, item_table, W1, b1, W2, b2, W3, b3):
    tab = _repack(user_table.T, item_table.T)
    ug, ig = _gather_kernel()(user.astype(jnp.int32), item.astype(jnp.int32),
                              tab)
    return _mlp(ug, ig, W1, b1, W2, b2, W3, b3)


# speedup vs baseline: 2.1027x; 1.0996x over previous
import functools

import jax
import jax.numpy as jnp
import numpy as np
from jax import lax
from jax.experimental import pallas as pl
from jax.experimental.pallas import tpu as pltpu
from jax.experimental.pallas import tpu_sc as plsc

B = 16384
D = 64
NV = 100000
NC, NS = 2, 16
NW = NC * NS
BPW = B // NW
RBLK = 2048
H = RBLK // 2
NB = (NV + RBLK - 1) // RBLK
NP = NB * RBLK
NROWS = NP // 2
NQ = 4 * NROWS

_HI = np.uint32(0xFFFF0000)


def _pack_words(x):
    xb = x.astype(jnp.bfloat16).astype(jnp.float32)
    bits = lax.bitcast_convert_type(xb, jnp.uint32)
    lo = lax.shift_right_logical(bits[0:32, :], jnp.uint32(16))
    hi = lax.bitwise_and(bits[32:64, :], _HI)
    return lax.bitcast_convert_type(lax.bitwise_or(hi, lo), jnp.float32)


def _repack_body(ut_ref, it_ref, o_ref):
    pu = jnp.transpose(_pack_words(ut_ref[...]), (1, 0))
    pi = jnp.transpose(_pack_words(it_ref[...]), (1, 0))
    o_ref[:, 0:32] = pu[0:H]
    o_ref[:, 32:64] = pi[0:H]
    o_ref[:, 64:96] = pu[H:]
    o_ref[:, 96:128] = pi[H:]


def _repack(ut_t, it_t):
    return pl.pallas_call(
        _repack_body,
        grid=(NB,),
        in_specs=[
            pl.BlockSpec((D, RBLK), lambda j: (0, j)),
            pl.BlockSpec((D, RBLK), lambda j: (0, j)),
        ],
        out_specs=pl.BlockSpec((H, 128), lambda j: (j, 0)),
        out_shape=jax.ShapeDtypeStruct((NROWS, 128), jnp.float32),
    )(ut_t, it_t)


def _remap(e, off):
    q = (lax.shift_left(lax.shift_right_logical(e, 11), 12)
         + lax.shift_left(lax.bitwise_and(e, 1023), 2)
         + lax.shift_left(lax.bitwise_and(lax.shift_right_logical(e, 10), 1), 1))
    return q + off


@functools.lru_cache(maxsize=None)
def _gather_kernel():
    mesh = plsc.VectorSubcoreMesh(core_axis_name="c", subcore_axis_name="s")

    @functools.partial(
        pl.kernel,
        out_type=(
            jax.ShapeDtypeStruct((B, 32), jnp.float32),
            jax.ShapeDtypeStruct((B, 32), jnp.float32),
        ),
        mesh=mesh,
        scratch_types=[
            pltpu.VMEM((BPW,), jnp.int32),
            pltpu.VMEM((BPW,), jnp.int32),
            pltpu.VMEM((BPW, 32), jnp.float32),
            pltpu.VMEM((BPW, 32), jnp.float32),
            pltpu.SemaphoreType.DMA,
            pltpu.SemaphoreType.DMA,
        ],
        compiler_params=pltpu.CompilerParams(use_tc_tiling_on_sc=False),
    )
    def gather(user_hbm, item_hbm, tab_hbm, uout_hbm, iout_hbm,
               uidx_v, iidx_v, urows_v, irows_v, usem, isem):
        wid = lax.axis_index("s") * NC + lax.axis_index("c")
        base = wid * BPW
        pltpu.sync_copy(user_hbm.at[pl.ds(base, BPW)], uidx_v)
        pltpu.sync_copy(item_hbm.at[pl.ds(base, BPW)], iidx_v)
        for t in range(BPW // 16):
            sl = pl.ds(16 * t, 16)
            uidx_v[sl] = _remap(uidx_v[sl], 0)
            iidx_v[sl] = _remap(iidx_v[sl], 1)
        cu = pltpu.async_copy(tab_hbm.at[uidx_v], urows_v, usem)
        ci = pltpu.async_copy(tab_hbm.at[iidx_v], irows_v, isem)
        cu.wait()
        ci.wait()
        pltpu.sync_copy(urows_v, uout_hbm.at[pl.ds(base, BPW)])
        pltpu.sync_copy(irows_v, iout_hbm.at[pl.ds(base, BPW)])

    return gather


def _unpack(p):
    bits = lax.bitcast_convert_type(p, jnp.uint32)
    lo = lax.bitcast_convert_type(lax.shift_left(bits, jnp.uint32(16)),
                                  jnp.float32)
    hi = lax.bitcast_convert_type(lax.bitwise_and(bits, _HI), jnp.float32)
    return jnp.concatenate([lo, hi], axis=1).astype(jnp.bfloat16)


def _mlp_body(u_ref, i_ref, w1u_ref, w1i_ref, b1_ref, w2_ref, b2_ref,
              w3_ref, b3_ref, o_ref):
    for k in range(4):
        sl = pl.ds(32 * k, 32)
        u = _unpack(u_ref[:, sl])
        i = _unpack(i_ref[:, sl])
        h = (jnp.dot(u, w1u_ref[...], preferred_element_type=jnp.float32)
             + jnp.dot(i, w1i_ref[...], preferred_element_type=jnp.float32)
             + b1_ref[...])
        h = jnp.maximum(h, 0.0).astype(jnp.bfloat16)
        h = (jnp.dot(h, w2_ref[...], preferred_element_type=jnp.float32)
             + b2_ref[...])
        h = jnp.maximum(h, 0.0)
        y = jnp.sum(h * w3_ref[...], axis=1, keepdims=True) + b3_ref[...]
        o_ref[:, pl.ds(k, 1)] = jax.nn.sigmoid(y)


def _mlp(ug4, ig4, W1, b1, W2, b2, W3, b3, block_b=1024):
    w1t = W1.T.astype(jnp.bfloat16)
    w1u, w1i = w1t[:D], w1t[D:]
    w2t = W2.T.astype(jnp.bfloat16)
    w3 = W3.reshape(1, 32)
    full = lambda shape: pl.BlockSpec(shape, lambda b: (0, 0))
    return pl.pallas_call(
        _mlp_body,
        grid=(B // 4 // block_b,),
        in_specs=[
            pl.BlockSpec((block_b, 128), lambda b: (b, 0)),
            pl.BlockSpec((block_b, 128), lambda b: (b, 0)),
            full((D, 64)),
            full((D, 64)),
            full((1, 64)),
            full((64, 32)),
            full((1, 32)),
            full((1, 32)),
            full((1, 1)),
        ],
        out_specs=pl.BlockSpec((block_b, 4), lambda b: (b, 0)),
        out_shape=jax.ShapeDtypeStruct((B // 4, 4), jnp.float32),
    )(ug4, ig4, w1u, w1i, b1.reshape(1, 64), w2t, b2.reshape(1, 32), w3,
      b3.reshape(1, 1))


def kernel(user, item, user_table, item_table, W1, b1, W2, b2, W3, b3):
    tab = _repack(user_table.T, item_table.T)
    tabq = tab.reshape(NQ, 32)
    ug, ig = _gather_kernel()(user.astype(jnp.int32), item.astype(jnp.int32),
                              tabq)
    y4 = _mlp(ug.reshape(B // 4, 128), ig.reshape(B // 4, 128),
              W1, b1, W2, b2, W3, b3)
    return y4.reshape(B, 1)


# speedup vs baseline: 2.7344x; 1.3004x over previous
import functools

import jax
import jax.numpy as jnp
import numpy as np
from jax import lax
from jax.experimental import pallas as pl
from jax.experimental.pallas import tpu as pltpu
from jax.experimental.pallas import tpu_sc as plsc

B = 16384
D = 64
NV = 100000
NC, NS = 2, 16
NW = NC * NS
BPW = B // NW
RBLK = 2048
H = RBLK // 2
NB = (NV + RBLK - 1) // RBLK
NP = NB * RBLK
NROWS = NP // 2
NQ = 4 * NROWS

_HI = np.uint32(0xFFFF0000)


def _pack_words(x):
    xb = x.astype(jnp.bfloat16).astype(jnp.float32)
    bits = lax.bitcast_convert_type(xb, jnp.uint32)
    lo = lax.shift_right_logical(bits[0:32, :], jnp.uint32(16))
    hi = lax.bitwise_and(bits[32:64, :], _HI)
    return lax.bitwise_or(hi, lo)


def _repack_body(ut_ref, it_ref, o_ref):
    pu = _pack_words(ut_ref[...])
    pi = _pack_words(it_ref[...])
    x = jnp.concatenate(
        [pu[:, 0:H], pi[:, 0:H], pu[:, H:], pi[:, H:]], axis=0)
    o_ref[...] = jnp.transpose(lax.bitcast_convert_type(x, jnp.float32), (1, 0))


def _repack(ut_t, it_t):
    return pl.pallas_call(
        _repack_body,
        grid=(NB,),
        in_specs=[
            pl.BlockSpec((D, RBLK), lambda j: (0, j)),
            pl.BlockSpec((D, RBLK), lambda j: (0, j)),
        ],
        out_specs=pl.BlockSpec((H, 128), lambda j: (j, 0)),
        out_shape=jax.ShapeDtypeStruct((NROWS, 128), jnp.float32),
    )(ut_t, it_t)


def _remap(e, off):
    q = (lax.shift_left(lax.shift_right_logical(e, 11), 12)
         + lax.shift_left(lax.bitwise_and(e, 1023), 2)
         + lax.shift_left(lax.bitwise_and(lax.shift_right_logical(e, 10), 1), 1))
    return q + off


@functools.lru_cache(maxsize=None)
def _gather_kernel():
    mesh = plsc.VectorSubcoreMesh(core_axis_name="c", subcore_axis_name="s")

    @functools.partial(
        pl.kernel,
        out_type=(
            jax.ShapeDtypeStruct((B, 32), jnp.float32),
            jax.ShapeDtypeStruct((B, 32), jnp.float32),
        ),
        mesh=mesh,
        scratch_types=[
            pltpu.VMEM((BPW,), jnp.int32),
            pltpu.VMEM((BPW,), jnp.int32),
            pltpu.VMEM((BPW, 32), jnp.float32),
            pltpu.VMEM((BPW, 32), jnp.float32),
            pltpu.SemaphoreType.DMA,
            pltpu.SemaphoreType.DMA,
        ],
        compiler_params=pltpu.CompilerParams(use_tc_tiling_on_sc=False),
    )
    def gather(user_hbm, item_hbm, tab_hbm, uout_hbm, iout_hbm,
               uidx_v, iidx_v, urows_v, irows_v, usem, isem):
        wid = lax.axis_index("s") * NC + lax.axis_index("c")
        base = wid * BPW
        pltpu.sync_copy(user_hbm.at[pl.ds(base, BPW)], uidx_v)
        pltpu.sync_copy(item_hbm.at[pl.ds(base, BPW)], iidx_v)
        for t in range(BPW // 16):
            sl = pl.ds(16 * t, 16)
            uidx_v[sl] = _remap(uidx_v[sl], 0)
            iidx_v[sl] = _remap(iidx_v[sl], 1)
        cu = pltpu.async_copy(tab_hbm.at[uidx_v], urows_v, usem)
        ci = pltpu.async_copy(tab_hbm.at[iidx_v], irows_v, isem)
        cu.wait()
        ci.wait()
        pltpu.sync_copy(urows_v, uout_hbm.at[pl.ds(base, BPW)])
        pltpu.sync_copy(irows_v, iout_hbm.at[pl.ds(base, BPW)])

    return gather


def _unpack_full(p):
    bits = lax.bitcast_convert_type(p, jnp.uint32)
    lo = lax.bitcast_convert_type(lax.shift_left(bits, jnp.uint32(16)),
                                  jnp.float32)
    hi = lax.bitcast_convert_type(lax.bitwise_and(bits, _HI), jnp.float32)
    return lo.astype(jnp.bfloat16), hi.astype(jnp.bfloat16)


def _mlp_body(u_ref, i_ref, wul_ref, wuh_ref, wil_ref, wih_ref, b1_ref,
              w2_ref, b2_ref, w3_ref, b3_ref, o_ref):
    ulo, uhi = _unpack_full(u_ref[...])
    ilo, ihi = _unpack_full(i_ref[...])
    h = (jnp.dot(ulo, wul_ref[...], preferred_element_type=jnp.float32)
         + jnp.dot(uhi, wuh_ref[...], preferred_element_type=jnp.float32)
         + jnp.dot(ilo, wil_ref[...], preferred_element_type=jnp.float32)
         + jnp.dot(ihi, wih_ref[...], preferred_element_type=jnp.float32)
         + b1_ref[...])
    h = jnp.maximum(h, 0.0).astype(jnp.bfloat16)
    h = (jnp.dot(h, w2_ref[...], preferred_element_type=jnp.float32)
         + b2_ref[...])
    h = jnp.maximum(h, 0.0).astype(jnp.bfloat16)
    y = jnp.dot(h, w3_ref[...], preferred_element_type=jnp.float32) + b3_ref[...]
    o_ref[...] = jax.nn.sigmoid(y)


def _mlp(ug4, ig4, W1, b1, W2, b2, W3, b3, block_b=1024):
    w1t = W1.T.astype(jnp.bfloat16)
    eye4 = jnp.eye(4, dtype=jnp.bfloat16)
    wul = jnp.kron(eye4, w1t[0:32])
    wuh = jnp.kron(eye4, w1t[32:64])
    wil = jnp.kron(eye4, w1t[64:96])
    wih = jnp.kron(eye4, w1t[96:128])
    w2d = jnp.kron(eye4, W2.T.astype(jnp.bfloat16))
    w3d = jnp.kron(eye4, W3.T.astype(jnp.bfloat16))
    b1c = jnp.tile(b1, 4).reshape(1, 256)
    b2c = jnp.tile(b2, 4).reshape(1, 128)
    b3c = jnp.broadcast_to(b3.reshape(1, 1), (1, 4))
    full = lambda shape: pl.BlockSpec(shape, lambda b: (0, 0))
    return pl.pallas_call(
        _mlp_body,
        grid=(B // 4 // block_b,),
        in_specs=[
            pl.BlockSpec((block_b, 128), lambda b: (b, 0)),
            pl.BlockSpec((block_b, 128), lambda b: (b, 0)),
            full((128, 256)),
            full((128, 256)),
            full((128, 256)),
            full((128, 256)),
            full((1, 256)),
            full((256, 128)),
            full((1, 128)),
            full((128, 4)),
            full((1, 4)),
        ],
        out_specs=pl.BlockSpec((block_b, 4), lambda b: (b, 0)),
        out_shape=jax.ShapeDtypeStruct((B // 4, 4), jnp.float32),
    )(ug4, ig4, wul, wuh, wil, wih, b1c, w2d, b2c, w3d, b3c)


def kernel(user, item, user_table, item_table, W1, b1, W2, b2, W3, b3):
    tab = _repack(user_table.T, item_table.T)
    tabq = tab.reshape(NQ, 32)
    ug, ig = _gather_kernel()(user.astype(jnp.int32), item.astype(jnp.int32),
                              tabq)
    y4 = _mlp(ug.reshape(B // 4, 128), ig.reshape(B // 4, 128),
              W1, b1, W2, b2, W3, b3)
    return y4.reshape(B, 1)


# speedup vs baseline: 3.2490x; 1.1882x over previous
import functools

import jax
import jax.numpy as jnp
import numpy as np
from jax import lax
from jax.experimental import pallas as pl
from jax.experimental.pallas import tpu as pltpu
from jax.experimental.pallas import tpu_sc as plsc

B = 16384
D = 64
NV = 100000
NC, NS = 2, 16
NW = NC * NS
BPW = B // NW
RBLK = 4096
H = RBLK // 2
NB = (NV + RBLK - 1) // RBLK
NP = NB * RBLK
NROWS = NP // 2
NQ = 4 * NROWS

_HI = np.uint32(0xFFFF0000)


def _pack_words(x):
    xb = x.astype(jnp.bfloat16).astype(jnp.float32)
    bits = lax.bitcast_convert_type(xb, jnp.uint32)
    lo = lax.shift_right_logical(bits[0:32, :], jnp.uint32(16))
    hi = lax.bitwise_and(bits[32:64, :], _HI)
    return lax.bitwise_or(hi, lo)


def _repack_body(ut_ref, it_ref, o_ref):
    pu = _pack_words(ut_ref[...])
    pi = _pack_words(it_ref[...])
    x = jnp.concatenate(
        [pu[:, 0:H], pi[:, 0:H], pu[:, H:], pi[:, H:]], axis=0)
    o_ref[...] = jnp.transpose(lax.bitcast_convert_type(x, jnp.float32), (1, 0))


def _repack(ut_t, it_t):
    return pl.pallas_call(
        _repack_body,
        grid=(NB,),
        in_specs=[
            pl.BlockSpec((D, RBLK), lambda j: (0, j)),
            pl.BlockSpec((D, RBLK), lambda j: (0, j)),
        ],
        out_specs=pl.BlockSpec((H, 128), lambda j: (j, 0)),
        out_shape=jax.ShapeDtypeStruct((NROWS, 128), jnp.float32),
    )(ut_t, it_t)


_JSH = RBLK.bit_length() - 1


def _remap(e, off):
    q = (lax.shift_left(lax.shift_right_logical(e, _JSH), _JSH + 1)
         + lax.shift_left(lax.bitwise_and(e, H - 1), 2)
         + lax.shift_left(lax.bitwise_and(lax.shift_right_logical(e, _JSH - 1), 1), 1))
    return q + off


@functools.lru_cache(maxsize=None)
def _gather_kernel():
    mesh = plsc.VectorSubcoreMesh(core_axis_name="c", subcore_axis_name="s")

    @functools.partial(
        pl.kernel,
        out_type=(
            jax.ShapeDtypeStruct((B, 32), jnp.float32),
            jax.ShapeDtypeStruct((B, 32), jnp.float32),
        ),
        mesh=mesh,
        scratch_types=[
            pltpu.VMEM((BPW,), jnp.int32),
            pltpu.VMEM((BPW,), jnp.int32),
            pltpu.VMEM((BPW, 32), jnp.float32),
            pltpu.VMEM((BPW, 32), jnp.float32),
            pltpu.SemaphoreType.DMA,
            pltpu.SemaphoreType.DMA,
        ],
        compiler_params=pltpu.CompilerParams(use_tc_tiling_on_sc=False),
    )
    def gather(user_hbm, item_hbm, tab_hbm, uout_hbm, iout_hbm,
               uidx_v, iidx_v, urows_v, irows_v, usem, isem):
        wid = lax.axis_index("s") * NC + lax.axis_index("c")
        base = wid * BPW
        pltpu.sync_copy(user_hbm.at[pl.ds(base, BPW)], uidx_v)
        pltpu.sync_copy(item_hbm.at[pl.ds(base, BPW)], iidx_v)
        for t in range(BPW // 16):
            sl = pl.ds(16 * t, 16)
            uidx_v[sl] = _remap(uidx_v[sl], 0)
            iidx_v[sl] = _remap(iidx_v[sl], 1)
        cu = pltpu.async_copy(tab_hbm.at[uidx_v], urows_v, usem)
        ci = pltpu.async_copy(tab_hbm.at[iidx_v], irows_v, isem)
        cu.wait()
        ci.wait()
        pltpu.sync_copy(urows_v, uout_hbm.at[pl.ds(base, BPW)])
        pltpu.sync_copy(irows_v, iout_hbm.at[pl.ds(base, BPW)])

    return gather


def _unpack_full(p):
    bits = lax.bitcast_convert_type(p, jnp.uint32)
    lo = lax.bitcast_convert_type(lax.shift_left(bits, jnp.uint32(16)),
                                  jnp.float32)
    hi = lax.bitcast_convert_type(lax.bitwise_and(bits, _HI), jnp.float32)
    return lo.astype(jnp.bfloat16), hi.astype(jnp.bfloat16)


def _mlp_body(u_ref, i_ref, wul_ref, wuh_ref, wil_ref, wih_ref, b1_ref,
              w2_ref, b2_ref, w3_ref, b3_ref, o_ref):
    ulo, uhi = _unpack_full(u_ref[...])
    ilo, ihi = _unpack_full(i_ref[...])
    h = (jnp.dot(ulo, wul_ref[...], preferred_element_type=jnp.float32)
         + jnp.dot(uhi, wuh_ref[...], preferred_element_type=jnp.float32)
         + jnp.dot(ilo, wil_ref[...], preferred_element_type=jnp.float32)
         + jnp.dot(ihi, wih_ref[...], preferred_element_type=jnp.float32)
         + b1_ref[...])
    h = jnp.maximum(h, 0.0).astype(jnp.bfloat16)
    h = (jnp.dot(h, w2_ref[...], preferred_element_type=jnp.float32)
         + b2_ref[...])
    h = jnp.maximum(h, 0.0).astype(jnp.bfloat16)
    y = jnp.dot(h, w3_ref[...], preferred_element_type=jnp.float32) + b3_ref[...]
    o_ref[...] = jnp.transpose(jax.nn.sigmoid(y), (1, 0))


def _mlp(ug4, ig4, W1, b1, W2, b2, W3, b3, block_b=1024):
    w1t = W1.T.astype(jnp.bfloat16)
    eye4 = jnp.eye(4, dtype=jnp.bfloat16)
    wul = jnp.kron(eye4, w1t[0:32])
    wuh = jnp.kron(eye4, w1t[32:64])
    wil = jnp.kron(eye4, w1t[64:96])
    wih = jnp.kron(eye4, w1t[96:128])
    w2d = jnp.kron(eye4, W2.T.astype(jnp.bfloat16))
    w3d = jnp.kron(eye4, W3.T.astype(jnp.bfloat16))
    b1c = jnp.tile(b1, 4).reshape(1, 256)
    b2c = jnp.tile(b2, 4).reshape(1, 128)
    b3c = jnp.broadcast_to(b3.reshape(1, 1), (1, 4))
    full = lambda shape: pl.BlockSpec(shape, lambda b: (0, 0))
    return pl.pallas_call(
        _mlp_body,
        grid=(B // 4 // block_b,),
        in_specs=[
            pl.BlockSpec((block_b, 128), lambda b: (b, 0)),
            pl.BlockSpec((block_b, 128), lambda b: (b, 0)),
            full((128, 256)),
            full((128, 256)),
            full((128, 256)),
            full((128, 256)),
            full((1, 256)),
            full((256, 128)),
            full((1, 128)),
            full((128, 4)),
            full((1, 4)),
        ],
        out_specs=pl.BlockSpec((4, block_b), lambda b: (0, b)),
        out_shape=jax.ShapeDtypeStruct((4, B // 4), jnp.float32),
    )(ug4, ig4, wul, wuh, wil, wih, b1c, w2d, b2c, w3d, b3c)


def kernel(user, item, user_table, item_table, W1, b1, W2, b2, W3, b3):
    tab = _repack(user_table.T, item_table.T)
    tabq = tab.reshape(NQ, 32)
    ug, ig = _gather_kernel()(user.astype(jnp.int32), item.astype(jnp.int32),
                              tabq)
    y4t = _mlp(ug.reshape(B // 4, 128), ig.reshape(B // 4, 128),
               W1, b1, W2, b2, W3, b3)
    return y4t.T.reshape(B, 1)


# speedup vs baseline: 3.3556x; 1.0328x over previous
import functools

import jax
import jax.numpy as jnp
import numpy as np
from jax import lax
from jax.experimental import pallas as pl
from jax.experimental.pallas import tpu as pltpu
from jax.experimental.pallas import tpu_sc as plsc

B = 16384
D = 64
NV = 100000
NC, NS = 2, 16
NW = NC * NS
BPW = B // NW
RBLK = 4096
H = RBLK // 2
NB = (NV + RBLK - 1) // RBLK
NP = NB * RBLK
NROWS = NP // 2
NQ = 4 * NROWS

_HI = np.uint32(0xFFFF0000)


def _pack_words(x):
    xb = x.astype(jnp.bfloat16).astype(jnp.float32)
    bits = lax.bitcast_convert_type(xb, jnp.uint32)
    lo = lax.shift_right_logical(bits[0:32, :], jnp.uint32(16))
    hi = lax.bitwise_and(bits[32:64, :], _HI)
    return lax.bitwise_or(hi, lo)


def _repack_body(ut_ref, it_ref, o_ref):
    pu = _pack_words(ut_ref[...])
    pi = _pack_words(it_ref[...])
    x = jnp.concatenate(
        [pu[:, 0:H], pi[:, 0:H], pu[:, H:], pi[:, H:]], axis=0)
    o_ref[...] = jnp.transpose(lax.bitcast_convert_type(x, jnp.float32), (1, 0))


def _repack(ut_t, it_t):
    return pl.pallas_call(
        _repack_body,
        grid=(NB,),
        in_specs=[
            pl.BlockSpec((D, RBLK), lambda j: (0, j)),
            pl.BlockSpec((D, RBLK), lambda j: (0, j)),
        ],
        out_specs=pl.BlockSpec((H, 128), lambda j: (j, 0)),
        out_shape=jax.ShapeDtypeStruct((NROWS, 128), jnp.float32),
    )(ut_t, it_t)


_JSH = RBLK.bit_length() - 1


def _remap(e, off):
    q = (lax.shift_left(lax.shift_right_logical(e, _JSH), _JSH + 1)
         + lax.shift_left(lax.bitwise_and(e, H - 1), 2)
         + lax.shift_left(lax.bitwise_and(lax.shift_right_logical(e, _JSH - 1), 1), 1))
    return q + off


@functools.lru_cache(maxsize=None)
def _gather_kernel():
    mesh = plsc.VectorSubcoreMesh(core_axis_name="c", subcore_axis_name="s")

    @functools.partial(
        pl.kernel,
        out_type=(
            jax.ShapeDtypeStruct((B // 4, 128), jnp.float32),
            jax.ShapeDtypeStruct((B // 4, 128), jnp.float32),
        ),
        mesh=mesh,
        scratch_types=[
            pltpu.VMEM((BPW,), jnp.int32),
            pltpu.VMEM((BPW,), jnp.int32),
            pltpu.VMEM((BPW, 32), jnp.float32),
            pltpu.VMEM((BPW, 32), jnp.float32),
            pltpu.SemaphoreType.DMA,
            pltpu.SemaphoreType.DMA,
        ],
        compiler_params=pltpu.CompilerParams(use_tc_tiling_on_sc=False),
    )
    def gather(user_hbm, item_hbm, tab_hbm, uout_hbm, iout_hbm,
               uidx_v, iidx_v, urows_v, irows_v, usem, isem):
        wid = lax.axis_index("s") * NC + lax.axis_index("c")
        base = wid * BPW
        kgrp = wid // (B // 4 // BPW)
        vbase = (wid % (B // 4 // BPW)) * BPW
        pltpu.sync_copy(user_hbm.at[pl.ds(base, BPW)], uidx_v)
        pltpu.sync_copy(item_hbm.at[pl.ds(base, BPW)], iidx_v)
        for t in range(BPW // 16):
            sl = pl.ds(16 * t, 16)
            uidx_v[sl] = _remap(uidx_v[sl], 0)
            iidx_v[sl] = _remap(iidx_v[sl], 1)
        cu = pltpu.async_copy(tab_hbm.at[uidx_v], urows_v, usem)
        ci = pltpu.async_copy(tab_hbm.at[iidx_v], irows_v, isem)
        cu.wait()
        ci.wait()
        pltpu.sync_copy(urows_v,
                        uout_hbm.at[pl.ds(vbase, BPW), pl.ds(32 * kgrp, 32)])
        pltpu.sync_copy(irows_v,
                        iout_hbm.at[pl.ds(vbase, BPW), pl.ds(32 * kgrp, 32)])

    return gather


def _unpack_full(p):
    bits = lax.bitcast_convert_type(p, jnp.uint32)
    lo = lax.bitcast_convert_type(lax.shift_left(bits, jnp.uint32(16)),
                                  jnp.float32)
    hi = lax.bitcast_convert_type(lax.bitwise_and(bits, _HI), jnp.float32)
    return lo.astype(jnp.bfloat16), hi.astype(jnp.bfloat16)


def _mlp_body(u_ref, i_ref, wul_ref, wuh_ref, wil_ref, wih_ref, b1_ref,
              w2_ref, b2_ref, w3_ref, b3_ref, o_ref):
    ulo, uhi = _unpack_full(u_ref[...])
    ilo, ihi = _unpack_full(i_ref[...])
    h = (jnp.dot(ulo, wul_ref[...], preferred_element_type=jnp.float32)
         + jnp.dot(uhi, wuh_ref[...], preferred_element_type=jnp.float32)
         + jnp.dot(ilo, wil_ref[...], preferred_element_type=jnp.float32)
         + jnp.dot(ihi, wih_ref[...], preferred_element_type=jnp.float32)
         + b1_ref[...])
    h = jnp.maximum(h, 0.0).astype(jnp.bfloat16)
    h = (jnp.dot(h, w2_ref[...], preferred_element_type=jnp.float32)
         + b2_ref[...])
    h = jnp.maximum(h, 0.0).astype(jnp.bfloat16)
    y = jnp.dot(h, w3_ref[...], preferred_element_type=jnp.float32) + b3_ref[...]
    o_ref[...] = jnp.transpose(jax.nn.sigmoid(y), (1, 0))


def _mlp(ug4, ig4, W1, b1, W2, b2, W3, b3, block_b=1024):
    w1t = W1.T.astype(jnp.bfloat16)
    eye4 = jnp.eye(4, dtype=jnp.bfloat16)
    wul = jnp.kron(eye4, w1t[0:32])
    wuh = jnp.kron(eye4, w1t[32:64])
    wil = jnp.kron(eye4, w1t[64:96])
    wih = jnp.kron(eye4, w1t[96:128])
    w2d = jnp.kron(eye4, W2.T.astype(jnp.bfloat16))
    w3d = jnp.kron(eye4, W3.T.astype(jnp.bfloat16))
    b1c = jnp.tile(b1, 4).reshape(1, 256)
    b2c = jnp.tile(b2, 4).reshape(1, 128)
    b3c = jnp.broadcast_to(b3.reshape(1, 1), (1, 4))
    full = lambda shape: pl.BlockSpec(shape, lambda b: (0, 0))
    return pl.pallas_call(
        _mlp_body,
        grid=(B // 4 // block_b,),
        in_specs=[
            pl.BlockSpec((block_b, 128), lambda b: (b, 0)),
            pl.BlockSpec((block_b, 128), lambda b: (b, 0)),
            full((128, 256)),
            full((128, 256)),
            full((128, 256)),
            full((128, 256)),
            full((1, 256)),
            full((256, 128)),
            full((1, 128)),
            full((128, 4)),
            full((1, 4)),
        ],
        out_specs=pl.BlockSpec((4, block_b), lambda b: (0, b)),
        out_shape=jax.ShapeDtypeStruct((4, B // 4), jnp.float32),
    )(ug4, ig4, wul, wuh, wil, wih, b1c, w2d, b2c, w3d, b3c)


def kernel(user, item, user_table, item_table, W1, b1, W2, b2, W3, b3):
    tab = _repack(user_table.T, item_table.T)
    tabq = tab.reshape(NQ, 32)
    ug, ig = _gather_kernel()(user.astype(jnp.int32), item.astype(jnp.int32),
                              tabq)
    y4t = _mlp(ug, ig, W1, b1, W2, b2, W3, b3)
    return y4t.reshape(B, 1)


# speedup vs baseline: 3.7023x; 1.1033x over previous
import functools

import jax
import jax.numpy as jnp
import numpy as np
from jax import lax
from jax.experimental import pallas as pl
from jax.experimental.pallas import tpu as pltpu
from jax.experimental.pallas import tpu_sc as plsc

B = 16384
D = 64
NV = 100000
NC, NS = 2, 16
NW = NC * NS
BPW = B // NW
RBLK = 8192
H = RBLK // 2
NB = (NV + RBLK - 1) // RBLK
NP = NB * RBLK
NROWS = NP // 2
NQ = 4 * NROWS

_HI = np.uint32(0xFFFF0000)


def _pack_words(x):
    xb = x.astype(jnp.bfloat16).astype(jnp.float32)
    bits = lax.bitcast_convert_type(xb, jnp.uint32)
    lo = lax.shift_right_logical(bits[0:32, :], jnp.uint32(16))
    hi = lax.bitwise_and(bits[32:64, :], _HI)
    return lax.bitwise_or(hi, lo)


def _repack_body(ut_ref, it_ref, o_ref):
    pu = _pack_words(ut_ref[...])
    pi = _pack_words(it_ref[...])
    x = jnp.concatenate(
        [pu[:, 0:H], pi[:, 0:H], pu[:, H:], pi[:, H:]], axis=0)
    o_ref[...] = jnp.transpose(lax.bitcast_convert_type(x, jnp.float32), (1, 0))


def _repack(ut_t, it_t):
    return pl.pallas_call(
        _repack_body,
        grid=(NB,),
        in_specs=[
            pl.BlockSpec((D, RBLK), lambda j: (0, j)),
            pl.BlockSpec((D, RBLK), lambda j: (0, j)),
        ],
        out_specs=pl.BlockSpec((H, 128), lambda j: (j, 0)),
        out_shape=jax.ShapeDtypeStruct((NROWS, 128), jnp.float32),
    )(ut_t, it_t)


_JSH = RBLK.bit_length() - 1


def _remap(e, off):
    q = (lax.shift_left(lax.shift_right_logical(e, _JSH), _JSH + 1)
         + lax.shift_left(lax.bitwise_and(e, H - 1), 2)
         + lax.shift_left(lax.bitwise_and(lax.shift_right_logical(e, _JSH - 1), 1), 1))
    return q + off


@functools.lru_cache(maxsize=None)
def _gather_kernel():
    mesh = plsc.VectorSubcoreMesh(core_axis_name="c", subcore_axis_name="s")

    @functools.partial(
        pl.kernel,
        out_type=(
            jax.ShapeDtypeStruct((B // 4, 128), jnp.float32),
            jax.ShapeDtypeStruct((B // 4, 128), jnp.float32),
        ),
        mesh=mesh,
        scratch_types=[
            pltpu.VMEM((BPW,), jnp.int32),
            pltpu.VMEM((BPW,), jnp.int32),
            pltpu.VMEM((BPW, 32), jnp.float32),
            pltpu.VMEM((BPW, 32), jnp.float32),
            pltpu.SemaphoreType.DMA,
            pltpu.SemaphoreType.DMA,
        ],
        compiler_params=pltpu.CompilerParams(use_tc_tiling_on_sc=False),
    )
    def gather(user_hbm, item_hbm, tab_hbm, uout_hbm, iout_hbm,
               uidx_v, iidx_v, urows_v, irows_v, usem, isem):
        wid = lax.axis_index("s") * NC + lax.axis_index("c")
        base = wid * BPW
        kgrp = wid // (B // 4 // BPW)
        vbase = (wid % (B // 4 // BPW)) * BPW
        pltpu.sync_copy(user_hbm.at[pl.ds(base, BPW)], uidx_v)
        pltpu.sync_copy(item_hbm.at[pl.ds(base, BPW)], iidx_v)
        for t in range(BPW // 16):
            sl = pl.ds(16 * t, 16)
            uidx_v[sl] = _remap(uidx_v[sl], 0)
            iidx_v[sl] = _remap(iidx_v[sl], 1)
        cu = pltpu.async_copy(tab_hbm.at[uidx_v], urows_v, usem)
        ci = pltpu.async_copy(tab_hbm.at[iidx_v], irows_v, isem)
        cu.wait()
        ci.wait()
        pltpu.sync_copy(urows_v,
                        uout_hbm.at[pl.ds(vbase, BPW), pl.ds(32 * kgrp, 32)])
        pltpu.sync_copy(irows_v,
                        iout_hbm.at[pl.ds(vbase, BPW), pl.ds(32 * kgrp, 32)])

    return gather


def _unpack_full(p):
    bits = lax.bitcast_convert_type(p, jnp.uint32)
    lo = lax.bitcast_convert_type(lax.shift_left(bits, jnp.uint32(16)),
                                  jnp.float32)
    hi = lax.bitcast_convert_type(lax.bitwise_and(bits, _HI), jnp.float32)
    return lo.astype(jnp.bfloat16), hi.astype(jnp.bfloat16)


def _mlp_body(u_ref, i_ref, wul_ref, wuh_ref, wil_ref, wih_ref, b1_ref,
              w2_ref, b2_ref, w3_ref, b3_ref, o_ref):
    ulo, uhi = _unpack_full(u_ref[...])
    ilo, ihi = _unpack_full(i_ref[...])
    h = (jnp.dot(ulo, wul_ref[...], preferred_element_type=jnp.float32)
         + jnp.dot(uhi, wuh_ref[...], preferred_element_type=jnp.float32)
         + jnp.dot(ilo, wil_ref[...], preferred_element_type=jnp.float32)
         + jnp.dot(ihi, wih_ref[...], preferred_element_type=jnp.float32)
         + b1_ref[...])
    h = jnp.maximum(h, 0.0).astype(jnp.bfloat16)
    h = (jnp.dot(h, w2_ref[...], preferred_element_type=jnp.float32)
         + b2_ref[...])
    h = jnp.maximum(h, 0.0).astype(jnp.bfloat16)
    y = jnp.dot(h, w3_ref[...], preferred_element_type=jnp.float32) + b3_ref[...]
    o_ref[...] = jnp.transpose(jax.nn.sigmoid(y), (1, 0))


def _mlp(ug4, ig4, W1, b1, W2, b2, W3, b3, block_b=2048):
    w1t = W1.T.astype(jnp.bfloat16)
    eye4 = jnp.eye(4, dtype=jnp.bfloat16)
    wul = jnp.kron(eye4, w1t[0:32])
    wuh = jnp.kron(eye4, w1t[32:64])
    wil = jnp.kron(eye4, w1t[64:96])
    wih = jnp.kron(eye4, w1t[96:128])
    w2d = jnp.kron(eye4, W2.T.astype(jnp.bfloat16))
    w3d = jnp.kron(eye4, W3.T.astype(jnp.bfloat16))
    b1c = jnp.tile(b1, 4).reshape(1, 256)
    b2c = jnp.tile(b2, 4).reshape(1, 128)
    b3c = jnp.broadcast_to(b3.reshape(1, 1), (1, 4))
    full = lambda shape: pl.BlockSpec(shape, lambda b: (0, 0))
    return pl.pallas_call(
        _mlp_body,
        grid=(B // 4 // block_b,),
        in_specs=[
            pl.BlockSpec((block_b, 128), lambda b: (b, 0)),
            pl.BlockSpec((block_b, 128), lambda b: (b, 0)),
            full((128, 256)),
            full((128, 256)),
            full((128, 256)),
            full((128, 256)),
            full((1, 256)),
            full((256, 128)),
            full((1, 128)),
            full((128, 4)),
            full((1, 4)),
        ],
        out_specs=pl.BlockSpec((4, block_b), lambda b: (0, b)),
        out_shape=jax.ShapeDtypeStruct((4, B // 4), jnp.float32),
    )(ug4, ig4, wul, wuh, wil, wih, b1c, w2d, b2c, w3d, b3c)


def kernel(user, item, user_table, item_table, W1, b1, W2, b2, W3, b3):
    tab = _repack(user_table.T, item_table.T)
    tabq = tab.reshape(NQ, 32)
    ug, ig = _gather_kernel()(user.astype(jnp.int32), item.astype(jnp.int32),
                              tabq)
    y4t = _mlp(ug, ig, W1, b1, W2, b2, W3, b3)
    return y4t.reshape(B, 1)


# speedup vs baseline: 3.7801x; 1.0210x over previous
import functools

import jax
import jax.numpy as jnp
import numpy as np
from jax import lax
from jax.experimental import pallas as pl
from jax.experimental.pallas import tpu as pltpu
from jax.experimental.pallas import tpu_sc as plsc

B = 16384
D = 64
NV = 100000
NC, NS = 2, 16
NW = NC * NS
BPW = B // NW
RBLK = 16384
H = RBLK // 2
NB = (NV + RBLK - 1) // RBLK
NP = NB * RBLK
NROWS = NP // 2
NQ = 4 * NROWS

_HI = np.uint32(0xFFFF0000)


def _pack_words(x):
    xb = x.astype(jnp.bfloat16).astype(jnp.float32)
    bits = lax.bitcast_convert_type(xb, jnp.uint32)
    lo = lax.shift_right_logical(bits[0:32, :], jnp.uint32(16))
    hi = lax.bitwise_and(bits[32:64, :], _HI)
    return lax.bitwise_or(hi, lo)


def _repack_body(ut_ref, it_ref, o_ref):
    pu = _pack_words(ut_ref[...])
    pi = _pack_words(it_ref[...])
    x = jnp.concatenate(
        [pu[:, 0:H], pi[:, 0:H], pu[:, H:], pi[:, H:]], axis=0)
    o_ref[...] = jnp.transpose(lax.bitcast_convert_type(x, jnp.float32), (1, 0))


def _repack(ut_t, it_t):
    return pl.pallas_call(
        _repack_body,
        grid=(NB,),
        in_specs=[
            pl.BlockSpec((D, RBLK), lambda j: (0, j)),
            pl.BlockSpec((D, RBLK), lambda j: (0, j)),
        ],
        out_specs=pl.BlockSpec((H, 128), lambda j: (j, 0)),
        out_shape=jax.ShapeDtypeStruct((NROWS, 128), jnp.float32),
    )(ut_t, it_t)


_JSH = RBLK.bit_length() - 1


def _remap(e, off):
    q = (lax.shift_left(lax.shift_right_logical(e, _JSH), _JSH + 1)
         + lax.shift_left(lax.bitwise_and(e, H - 1), 2)
         + lax.shift_left(lax.bitwise_and(lax.shift_right_logical(e, _JSH - 1), 1), 1))
    return q + off


@functools.lru_cache(maxsize=None)
def _gather_kernel():
    mesh = plsc.VectorSubcoreMesh(core_axis_name="c", subcore_axis_name="s")

    @functools.partial(
        pl.kernel,
        out_type=(
            jax.ShapeDtypeStruct((B // 4, 128), jnp.float32),
            jax.ShapeDtypeStruct((B // 4, 128), jnp.float32),
        ),
        mesh=mesh,
        scratch_types=[
            pltpu.VMEM((BPW,), jnp.int32),
            pltpu.VMEM((BPW,), jnp.int32),
            pltpu.VMEM((BPW, 32), jnp.float32),
            pltpu.VMEM((BPW, 32), jnp.float32),
            pltpu.SemaphoreType.DMA,
            pltpu.SemaphoreType.DMA,
        ],
        compiler_params=pltpu.CompilerParams(use_tc_tiling_on_sc=False),
    )
    def gather(user_hbm, item_hbm, tab_hbm, uout_hbm, iout_hbm,
               uidx_v, iidx_v, urows_v, irows_v, usem, isem):
        wid = lax.axis_index("s") * NC + lax.axis_index("c")
        base = wid * BPW
        kgrp = wid // (B // 4 // BPW)
        vbase = (wid % (B // 4 // BPW)) * BPW
        pltpu.sync_copy(user_hbm.at[pl.ds(base, BPW)], uidx_v)
        pltpu.sync_copy(item_hbm.at[pl.ds(base, BPW)], iidx_v)
        for t in range(BPW // 16):
            sl = pl.ds(16 * t, 16)
            uidx_v[sl] = _remap(uidx_v[sl], 0)
            iidx_v[sl] = _remap(iidx_v[sl], 1)
        cu = pltpu.async_copy(tab_hbm.at[uidx_v], urows_v, usem)
        ci = pltpu.async_copy(tab_hbm.at[iidx_v], irows_v, isem)
        cu.wait()
        ci.wait()
        pltpu.sync_copy(urows_v,
                        uout_hbm.at[pl.ds(vbase, BPW), pl.ds(32 * kgrp, 32)])
        pltpu.sync_copy(irows_v,
                        iout_hbm.at[pl.ds(vbase, BPW), pl.ds(32 * kgrp, 32)])

    return gather


def _unpack_full(p):
    bits = lax.bitcast_convert_type(p, jnp.uint32)
    lo = lax.bitcast_convert_type(lax.shift_left(bits, jnp.uint32(16)),
                                  jnp.float32)
    hi = lax.bitcast_convert_type(lax.bitwise_and(bits, _HI), jnp.float32)
    return lo.astype(jnp.bfloat16), hi.astype(jnp.bfloat16)


def _mlp_body(u_ref, i_ref, wul_ref, wuh_ref, wil_ref, wih_ref, b1_ref,
              w2_ref, b2_ref, w3_ref, b3_ref, o_ref):
    ulo, uhi = _unpack_full(u_ref[...])
    ilo, ihi = _unpack_full(i_ref[...])
    h = (jnp.dot(ulo, wul_ref[...], preferred_element_type=jnp.float32)
         + jnp.dot(uhi, wuh_ref[...], preferred_element_type=jnp.float32)
         + jnp.dot(ilo, wil_ref[...], preferred_element_type=jnp.float32)
         + jnp.dot(ihi, wih_ref[...], preferred_element_type=jnp.float32)
         + b1_ref[...])
    h = jnp.maximum(h, 0.0).astype(jnp.bfloat16)
    h = (jnp.dot(h, w2_ref[...], preferred_element_type=jnp.float32)
         + b2_ref[...])
    h = jnp.maximum(h, 0.0).astype(jnp.bfloat16)
    y = jnp.dot(h, w3_ref[...], preferred_element_type=jnp.float32) + b3_ref[...]
    o_ref[...] = jnp.transpose(jax.nn.sigmoid(y), (1, 0))


def _mlp(ug4, ig4, W1, b1, W2, b2, W3, b3, block_b=2048):
    w1t = W1.T.astype(jnp.bfloat16)
    eye4 = jnp.eye(4, dtype=jnp.bfloat16)
    wul = jnp.kron(eye4, w1t[0:32])
    wuh = jnp.kron(eye4, w1t[32:64])
    wil = jnp.kron(eye4, w1t[64:96])
    wih = jnp.kron(eye4, w1t[96:128])
    w2d = jnp.kron(eye4, W2.T.astype(jnp.bfloat16))
    w3d = jnp.kron(eye4, W3.T.astype(jnp.bfloat16))
    b1c = jnp.tile(b1, 4).reshape(1, 256)
    b2c = jnp.tile(b2, 4).reshape(1, 128)
    b3c = jnp.broadcast_to(b3.reshape(1, 1), (1, 4))
    full = lambda shape: pl.BlockSpec(shape, lambda b: (0, 0))
    return pl.pallas_call(
        _mlp_body,
        grid=(B // 4 // block_b,),
        in_specs=[
            pl.BlockSpec((block_b, 128), lambda b: (b, 0)),
            pl.BlockSpec((block_b, 128), lambda b: (b, 0)),
            full((128, 256)),
            full((128, 256)),
            full((128, 256)),
            full((128, 256)),
            full((1, 256)),
            full((256, 128)),
            full((1, 128)),
            full((128, 4)),
            full((1, 4)),
        ],
        out_specs=pl.BlockSpec((4, block_b), lambda b: (0, b)),
        out_shape=jax.ShapeDtypeStruct((4, B // 4), jnp.float32),
    )(ug4, ig4, wul, wuh, wil, wih, b1c, w2d, b2c, w3d, b3c)


def kernel(user, item, user_table, item_table, W1, b1, W2, b2, W3, b3):
    tab = _repack(user_table.T, item_table.T)
    tabq = tab.reshape(NQ, 32)
    ug, ig = _gather_kernel()(user.astype(jnp.int32), item.astype(jnp.int32),
                              tabq)
    y4t = _mlp(ug, ig, W1, b1, W2, b2, W3, b3)
    return y4t.reshape(B, 1)


# speedup vs baseline: 3.7819x; 1.0005x over previous
import functools

import jax
import jax.numpy as jnp
import numpy as np
from jax import lax
from jax.experimental import pallas as pl
from jax.experimental.pallas import tpu as pltpu
from jax.experimental.pallas import tpu_sc as plsc

B = 16384
D = 64
NV = 100000
NC, NS = 2, 16
NW = NC * NS
BPW = B // NW
RBLK = 16384
H = RBLK // 2
NB = (NV + RBLK - 1) // RBLK
NP = NB * RBLK
NROWS = NP // 2
NQ = 4 * NROWS

_HI = np.uint32(0xFFFF0000)


def _pack_words(x):
    xb = x.astype(jnp.bfloat16).astype(jnp.float32)
    bits = lax.bitcast_convert_type(xb, jnp.uint32)
    lo = lax.shift_right_logical(bits[0:32, :], jnp.uint32(16))
    hi = lax.bitwise_and(bits[32:64, :], _HI)
    return lax.bitwise_or(hi, lo)


def _repack_body(ut_ref, it_ref, o_ref):
    pu = _pack_words(ut_ref[...])
    pi = _pack_words(it_ref[...])
    x = jnp.concatenate(
        [pu[:, 0:H], pi[:, 0:H], pu[:, H:], pi[:, H:]], axis=0)
    o_ref[...] = jnp.transpose(lax.bitcast_convert_type(x, jnp.float32), (1, 0))


def _repack(ut_t, it_t):
    return pl.pallas_call(
        _repack_body,
        grid=(NB,),
        in_specs=[
            pl.BlockSpec((D, RBLK), lambda j: (0, j)),
            pl.BlockSpec((D, RBLK), lambda j: (0, j)),
        ],
        out_specs=pl.BlockSpec((H, 128), lambda j: (j, 0)),
        out_shape=jax.ShapeDtypeStruct((NROWS, 128), jnp.float32),
    )(ut_t, it_t)


_JSH = RBLK.bit_length() - 1


def _remap(e, off):
    q = (lax.shift_left(lax.shift_right_logical(e, _JSH), _JSH + 1)
         + lax.shift_left(lax.bitwise_and(e, H - 1), 2)
         + lax.shift_left(lax.bitwise_and(lax.shift_right_logical(e, _JSH - 1), 1), 1))
    return q + off


@functools.lru_cache(maxsize=None)
def _gather_kernel():
    mesh = plsc.VectorSubcoreMesh(core_axis_name="c", subcore_axis_name="s")

    @functools.partial(
        pl.kernel,
        out_type=(
            jax.ShapeDtypeStruct((B // 4, 128), jnp.float32),
            jax.ShapeDtypeStruct((B // 4, 128), jnp.float32),
        ),
        mesh=mesh,
        scratch_types=[
            pltpu.VMEM((BPW,), jnp.int32),
            pltpu.VMEM((BPW,), jnp.int32),
            pltpu.VMEM((BPW, 32), jnp.float32),
            pltpu.VMEM((BPW, 32), jnp.float32),
            pltpu.SemaphoreType.DMA,
            pltpu.SemaphoreType.DMA,
        ],
        compiler_params=pltpu.CompilerParams(use_tc_tiling_on_sc=False),
    )
    def gather(user_hbm, item_hbm, tab_hbm, uout_hbm, iout_hbm,
               uidx_v, iidx_v, urows_v, irows_v, usem, isem):
        wid = lax.axis_index("s") * NC + lax.axis_index("c")
        base = wid * BPW
        kgrp = wid // (B // 4 // BPW)
        vbase = (wid % (B // 4 // BPW)) * BPW
        pltpu.sync_copy(user_hbm.at[pl.ds(base, BPW)], uidx_v)
        pltpu.sync_copy(item_hbm.at[pl.ds(base, BPW)], iidx_v)
        for t in range(BPW // 16):
            sl = pl.ds(16 * t, 16)
            uidx_v[sl] = _remap(uidx_v[sl], 0)
            iidx_v[sl] = _remap(iidx_v[sl], 1)
        cu = pltpu.async_copy(tab_hbm.at[uidx_v], urows_v, usem)
        ci = pltpu.async_copy(tab_hbm.at[iidx_v], irows_v, isem)
        cu.wait()
        ci.wait()
        pltpu.sync_copy(urows_v,
                        uout_hbm.at[pl.ds(vbase, BPW), pl.ds(32 * kgrp, 32)])
        pltpu.sync_copy(irows_v,
                        iout_hbm.at[pl.ds(vbase, BPW), pl.ds(32 * kgrp, 32)])

    return gather


def _unpack_full(p):
    bits = lax.bitcast_convert_type(p, jnp.uint32)
    lo = lax.bitcast_convert_type(lax.shift_left(bits, jnp.uint32(16)),
                                  jnp.float32)
    hi = lax.bitcast_convert_type(lax.bitwise_and(bits, _HI), jnp.float32)
    return lo.astype(jnp.bfloat16), hi.astype(jnp.bfloat16)


def _mlp_body(u_ref, i_ref, wul_ref, wuh_ref, wil_ref, wih_ref, b1_ref,
              w2_ref, b2_ref, w3_ref, b3_ref, o_ref):
    ulo, uhi = _unpack_full(u_ref[...])
    ilo, ihi = _unpack_full(i_ref[...])
    h = (jnp.dot(ulo, wul_ref[...], preferred_element_type=jnp.float32)
         + jnp.dot(uhi, wuh_ref[...], preferred_element_type=jnp.float32)
         + jnp.dot(ilo, wil_ref[...], preferred_element_type=jnp.float32)
         + jnp.dot(ihi, wih_ref[...], preferred_element_type=jnp.float32)
         + b1_ref[...])
    h = jnp.maximum(h, 0.0).astype(jnp.bfloat16)
    h = (jnp.dot(h, w2_ref[...], preferred_element_type=jnp.float32)
         + b2_ref[...])
    h = jnp.maximum(h, 0.0).astype(jnp.bfloat16)
    y = jnp.dot(h, w3_ref[...], preferred_element_type=jnp.float32) + b3_ref[...]
    o_ref[...] = jnp.transpose(jax.nn.sigmoid(y), (1, 0))


def _mlp(ug4, ig4, W1, b1, W2, b2, W3, b3, block_b=4096):
    w1t = W1.T.astype(jnp.bfloat16)
    eye4 = jnp.eye(4, dtype=jnp.bfloat16)
    wul = jnp.kron(eye4, w1t[0:32])
    wuh = jnp.kron(eye4, w1t[32:64])
    wil = jnp.kron(eye4, w1t[64:96])
    wih = jnp.kron(eye4, w1t[96:128])
    w2d = jnp.kron(eye4, W2.T.astype(jnp.bfloat16))
    w3d = jnp.kron(eye4, W3.T.astype(jnp.bfloat16))
    b1c = jnp.tile(b1, 4).reshape(1, 256)
    b2c = jnp.tile(b2, 4).reshape(1, 128)
    b3c = jnp.broadcast_to(b3.reshape(1, 1), (1, 4))
    full = lambda shape: pl.BlockSpec(shape, lambda b: (0, 0))
    return pl.pallas_call(
        _mlp_body,
        grid=(B // 4 // block_b,),
        in_specs=[
            pl.BlockSpec((block_b, 128), lambda b: (b, 0)),
            pl.BlockSpec((block_b, 128), lambda b: (b, 0)),
            full((128, 256)),
            full((128, 256)),
            full((128, 256)),
            full((128, 256)),
            full((1, 256)),
            full((256, 128)),
            full((1, 128)),
            full((128, 4)),
            full((1, 4)),
        ],
        out_specs=pl.BlockSpec((4, block_b), lambda b: (0, b)),
        out_shape=jax.ShapeDtypeStruct((4, B // 4), jnp.float32),
    )(ug4, ig4, wul, wuh, wil, wih, b1c, w2d, b2c, w3d, b3c)


def kernel(user, item, user_table, item_table, W1, b1, W2, b2, W3, b3):
    tab = _repack(user_table.T, item_table.T)
    tabq = tab.reshape(NQ, 32)
    ug, ig = _gather_kernel()(user.astype(jnp.int32), item.astype(jnp.int32),
                              tabq)
    y4t = _mlp(ug, ig, W1, b1, W2, b2, W3, b3)
    return y4t.reshape(B, 1)
